# Initial kernel scaffold; baseline (speedup 1.0000x reference)
import jax
import jax.numpy as jnp
from jax.experimental import pallas as pl


def kernel(o, m, c, edge_index_o, edge_index_m, edge_index_c, W1_rel, W1_root, b1, W2_rel, W2_root, b2, W3_rel, W3_root, b3, W4_rel, W4_root, b4):
    raise NotImplementedError("write your pallas kernel here")


# speedup vs baseline: 9.7063x; 9.7063x over previous
import functools

import jax
import jax.numpy as jnp
from jax import lax
from jax.experimental import pallas as pl
from jax.experimental.pallas import tpu as pltpu
from jax.experimental.pallas import tpu_sc as plsc

NN = 100000
NP = 100352
EE = 3200000
RC = 25088
EPG = RC * 128
NW = 32
BLK = 8
NBLK = RC // (NW * BLK)
CC = 1568
RPT = NP // 16
RB = 2048
GC = NP // RB

_SC_PARAMS = pltpu.CompilerParams(use_tc_tiling_on_sc=False)


def _zero_acc(sid, acc, bounce):
    def body(i, carry):
        pltpu.sync_copy(bounce, acc.at[pl.ds(sid * RPT + i * CC, CC)])
        return carry

    lax.fori_loop(0, RPT // CC, body, 0)


def _run_phase(cid, sid, wid, src_hbm, dst_hbm, x_hbm, acc, out,
               si, di, rows, bounce, zbuf, sem):
    base = wid * NBLK * BLK

    def body(b, carry):
        r = base + b * BLK
        for j in range(BLK):
            pltpu.sync_copy(src_hbm.at[r + j], si)
            pltpu.sync_copy(dst_hbm.at[r + j], di)
            pltpu.async_copy(x_hbm.at[si], rows.at[j], sem).wait()
            pltpu.sync_copy(rows.at[j], acc.at[di], add=True)
        return carry

    lax.fori_loop(0, NBLK, body, 0)
    plsc.subcore_barrier()

    def obody(i, carry):
        off = sid * RPT + i * CC
        pltpu.sync_copy(acc.at[pl.ds(off, CC)], bounce)
        pltpu.sync_copy(bounce, out.at[cid].at[pl.ds(off, CC)])
        return carry

    lax.fori_loop(0, RPT // CC, obody, 0)
    _zero_acc(sid, acc, zbuf)
    plsc.subcore_barrier()


def _make_seg(mesh, w, n_phases):
    outs = tuple(jax.ShapeDtypeStruct((2, NP, w), jnp.float32)
                 for _ in range(n_phases))

    @functools.partial(
        pl.kernel,
        out_type=outs,
        mesh=mesh,
        compiler_params=_SC_PARAMS,
        scratch_types=[
            pltpu.VMEM((128,), jnp.int32),
            pltpu.VMEM((128,), jnp.int32),
            pltpu.VMEM((BLK, 128, w), jnp.float32),
            pltpu.VMEM((CC, w), jnp.float32),
            pltpu.VMEM((CC, w), jnp.float32),
            pltpu.VMEM_SHARED((NP, w), jnp.float32),
            pltpu.SemaphoreType.DMA,
        ],
    )
    def seg(*refs):
        tables = refs[0:n_phases]
        srcs = refs[n_phases:2 * n_phases]
        dsts = refs[2 * n_phases:3 * n_phases]
        zz = refs[3 * n_phases]
        outs_ = refs[3 * n_phases + 1:4 * n_phases + 1]
        si, di, rows, bounce, zbuf, acc, sem = refs[4 * n_phases + 1:]
        cid = lax.axis_index("c")
        sid = lax.axis_index("s")
        wid = sid * 2 + cid
        pltpu.sync_copy(zz, zbuf)
        _zero_acc(sid, acc, zbuf)
        plsc.subcore_barrier()
        for p in range(n_phases):
            _run_phase(cid, sid, wid, srcs[p], dsts[p], tables[p], acc,
                       outs_[p], si, di, rows, bounce, zbuf, sem)

    return seg


@functools.cache
def _sc_kernels():
    mesh = plsc.VectorSubcoreMesh(core_axis_name="c", subcore_axis_name="s",
                                  num_cores=2, num_subcores=16)
    seg3x4 = _make_seg(mesh, 8, 3)
    seg2x8 = _make_seg(mesh, 8, 2)
    return seg3x4, seg2x8


def _elu(h):
    return jnp.where(h > 0, h, jnp.exp(jnp.minimum(h, 0.0)) - 1.0)


def _h1_body(p_ref, x_ref, wr_ref, wo_ref, b_ref, o_ref):
    agg = p_ref[0] + p_ref[1]
    h = (jnp.dot(agg[:, :2], wr_ref[...], preferred_element_type=jnp.float32)
         + jnp.dot(x_ref[...], wo_ref[...], preferred_element_type=jnp.float32)
         + b_ref[...])
    o_ref[...] = _elu(h)


_h1_call = pl.pallas_call(
    _h1_body,
    grid=(GC,),
    in_specs=[
        pl.BlockSpec((2, RB, 8), lambda i: (0, i, 0)),
        pl.BlockSpec((RB, 2), lambda i: (i, 0)),
        pl.BlockSpec((2, 8), lambda i: (0, 0)),
        pl.BlockSpec((2, 8), lambda i: (0, 0)),
        pl.BlockSpec((1, 8), lambda i: (0, 0)),
    ],
    out_specs=pl.BlockSpec((RB, 8), lambda i: (i, 0)),
    out_shape=jax.ShapeDtypeStruct((NP, 8), jnp.float32),
)


def _h3_body(p_ref, x_ref, wr_ref, wo_ref, b_ref, oa_ref, ob_ref):
    agg = p_ref[0] + p_ref[1]
    h = (jnp.dot(agg[:, :2], wr_ref[...], preferred_element_type=jnp.float32)
         + jnp.dot(x_ref[...], wo_ref[...], preferred_element_type=jnp.float32)
         + b_ref[...])
    h = _elu(h)
    oa_ref[...] = h[:, :8]
    ob_ref[...] = h[:, 8:]


_h3_call = pl.pallas_call(
    _h3_body,
    grid=(GC,),
    in_specs=[
        pl.BlockSpec((2, RB, 8), lambda i: (0, i, 0)),
        pl.BlockSpec((RB, 2), lambda i: (i, 0)),
        pl.BlockSpec((2, 16), lambda i: (0, 0)),
        pl.BlockSpec((2, 16), lambda i: (0, 0)),
        pl.BlockSpec((1, 16), lambda i: (0, 0)),
    ],
    out_specs=[
        pl.BlockSpec((RB, 8), lambda i: (i, 0)),
        pl.BlockSpec((RB, 8), lambda i: (i, 0)),
    ],
    out_shape=[
        jax.ShapeDtypeStruct((NP, 8), jnp.float32),
        jax.ShapeDtypeStruct((NP, 8), jnp.float32),
    ],
)


def _pool_body(p_ref, h1_ref, wr_ref, wo_ref, b_ref, o_ref):
    i = pl.program_id(0)
    agg = p_ref[0] + p_ref[1]
    h2 = (jnp.dot(agg, wr_ref[...], preferred_element_type=jnp.float32)
          + jnp.dot(h1_ref[...], wo_ref[...], preferred_element_type=jnp.float32)
          + b_ref[...])
    rows = lax.broadcasted_iota(jnp.int32, (RB, 16), 0) + i * RB
    h2 = jnp.where(rows < NN, h2, -jnp.inf)
    bm = jnp.max(h2, axis=0, keepdims=True)

    @pl.when(i == 0)
    def _():
        o_ref[...] = bm

    @pl.when(i > 0)
    def _():
        o_ref[...] = jnp.maximum(o_ref[...], bm)


_pool_call = pl.pallas_call(
    _pool_body,
    grid=(GC,),
    in_specs=[
        pl.BlockSpec((2, RB, 8), lambda i: (0, i, 0)),
        pl.BlockSpec((RB, 8), lambda i: (i, 0)),
        pl.BlockSpec((8, 16), lambda i: (0, 0)),
        pl.BlockSpec((8, 16), lambda i: (0, 0)),
        pl.BlockSpec((1, 16), lambda i: (0, 0)),
    ],
    out_specs=pl.BlockSpec((1, 16), lambda i: (0, 0)),
    out_shape=jax.ShapeDtypeStruct((1, 16), jnp.float32),
)


def _final_body(p4a_ref, p4b_ref, h3a_ref, h3b_ref, pc4_ref, c_ref,
                of_ref, mf_ref, wr_ref, wo_ref, b_ref, o_ref):
    a4 = jnp.concatenate([p4a_ref[0] + p4a_ref[1],
                          p4b_ref[0] + p4b_ref[1]], axis=1)
    h3 = jnp.concatenate([h3a_ref[...], h3b_ref[...]], axis=1)
    deg = pc4_ref[0][:, 2:3] + pc4_ref[1][:, 2:3]
    wr = wr_ref[...]
    wo = wo_ref[...]
    of = of_ref[...]
    mf = mf_ref[...]
    pr_rel = (jnp.dot(of, wr[16:32], preferred_element_type=jnp.float32)
              + jnp.dot(mf, wr[32:48], preferred_element_type=jnp.float32))
    pr_root = (jnp.dot(of, wo[16:32], preferred_element_type=jnp.float32)
               + jnp.dot(mf, wo[32:48], preferred_element_type=jnp.float32))
    z = (jnp.dot(a4, wr[:16], preferred_element_type=jnp.float32)
         + jnp.dot(h3, wo[:16], preferred_element_type=jnp.float32)
         + deg * pr_rel + pr_root + b_ref[...])
    o_ref[...] = jax.nn.sigmoid(z) + c_ref[...]


_final_call = pl.pallas_call(
    _final_body,
    grid=(GC,),
    in_specs=[
        pl.BlockSpec((2, RB, 8), lambda i: (0, i, 0)),
        pl.BlockSpec((2, RB, 8), lambda i: (0, i, 0)),
        pl.BlockSpec((RB, 8), lambda i: (i, 0)),
        pl.BlockSpec((RB, 8), lambda i: (i, 0)),
        pl.BlockSpec((2, RB, 8), lambda i: (0, i, 0)),
        pl.BlockSpec((RB, 2), lambda i: (i, 0)),
        pl.BlockSpec((1, 16), lambda i: (0, 0)),
        pl.BlockSpec((1, 16), lambda i: (0, 0)),
        pl.BlockSpec((48, 2), lambda i: (0, 0)),
        pl.BlockSpec((48, 2), lambda i: (0, 0)),
        pl.BlockSpec((1, 2), lambda i: (0, 0)),
    ],
    out_specs=pl.BlockSpec((RB, 2), lambda i: (i, 0)),
    out_shape=jax.ShapeDtypeStruct((NP, 2), jnp.float32),
)


def _pad_idx(v):
    pad = EPG - EE
    fill = (jnp.arange(pad, dtype=jnp.int32) % 128) + NN
    return jnp.concatenate([v, fill]).reshape(RC, 128)


def _pad_idx3(v):
    return _pad_idx(v).reshape(RC, 1, 128)


def _pack8(x):
    one = jnp.ones((NN, 1), jnp.float32)
    return jnp.pad(jnp.concatenate([x, one], axis=1), ((0, NP - NN), (0, 5)))


def _padn(x):
    return jnp.pad(x, ((0, NP - NN), (0, 0)))


def kernel(o, m, c, edge_index_o, edge_index_m, edge_index_c,
           W1_rel, W1_root, b1, W2_rel, W2_root, b2,
           W3_rel, W3_root, b3, W4_rel, W4_root, b4):
    so, do_ = _pad_idx(edge_index_o[0]), _pad_idx(edge_index_o[1])
    sm, dm = _pad_idx(edge_index_m[0]), _pad_idx(edge_index_m[1])
    sc_, dc = _pad_idx(edge_index_c[0]), _pad_idx(edge_index_c[1])

    o_pad, m_pad, c_pad = _padn(o), _padn(m), _padn(c)
    z8 = jnp.zeros((CC, 8), jnp.float32)

    seg3x4, seg2x8 = _sc_kernels()
    p_o4, p_m4, p_c4 = seg3x4(_pack8(o), _pack8(m), _pack8(c),
                              so, sm, sc_, do_, dm, dc, z8)
    h1o = _h1_call(p_o4, o_pad, W1_rel, W1_root, b1.reshape(1, 8))
    h1m = _h1_call(p_m4, m_pad, W1_rel, W1_root, b1.reshape(1, 8))
    h3a, h3b = _h3_call(p_c4, c_pad, W3_rel, W3_root, b3.reshape(1, 16))

    p2o, p2m = seg2x8(h1o, h1m, so, sm, do_, dm, z8)
    p4a, p4b = seg2x8(h3a, h3b, sc_, sc_, dc, dc, z8)

    o_feat = _pool_call(p2o, h1o, W2_rel, W2_root, b2.reshape(1, 16))
    m_feat = _pool_call(p2m, h1m, W2_rel, W2_root, b2.reshape(1, 16))

    out = _final_call(p4a, p4b, h3a, h3b, p_c4, c_pad, o_feat, m_feat,
                      W4_rel, W4_root, b4.reshape(1, 2))
    return out[:NN]


# speedup vs baseline: 35.2180x; 3.6284x over previous
import functools

import jax
import jax.numpy as jnp
from jax import lax
from jax.experimental import pallas as pl
from jax.experimental.pallas import tpu as pltpu
from jax.experimental.pallas import tpu_sc as plsc

NN = 100000
NP = 100352
EE = 3200000
RC = 25088
EPG = RC * 128
NW = 32
CH = 2048
NCH = EPG // (NW * CH)
CC = 1568
RPT = NP // 16
RB = 2048
GC = NP // RB

_SC_PARAMS = pltpu.CompilerParams(use_tc_tiling_on_sc=False)


def _zero_acc(sid, acc, bounce):
    def body(i, carry):
        pltpu.sync_copy(bounce, acc.at[pl.ds(sid * RPT + i * CC, CC)])
        return carry

    lax.fori_loop(0, RPT // CC, body, 0)


def _run_phase(cid, sid, wid, src_hbm, dst_hbm, x_hbm, acc, out,
               si, di, rows, bounce, zbuf, sem):
    base = wid * NCH * CH

    def body(b, carry):
        r = base + b * CH
        pltpu.sync_copy(src_hbm.at[pl.ds(r, CH)], si)
        pltpu.sync_copy(dst_hbm.at[pl.ds(r, CH)], di)
        pltpu.async_copy(x_hbm.at[si], rows, sem).wait()
        pltpu.sync_copy(rows, acc.at[di], add=True)
        return carry

    lax.fori_loop(0, NCH, body, 0)
    plsc.subcore_barrier()

    def obody(i, carry):
        off = sid * RPT + i * CC
        pltpu.sync_copy(acc.at[pl.ds(off, CC)], bounce)
        pltpu.sync_copy(bounce, out.at[cid].at[pl.ds(off, CC)])
        return carry

    lax.fori_loop(0, RPT // CC, obody, 0)
    _zero_acc(sid, acc, zbuf)
    plsc.subcore_barrier()


def _make_seg(mesh, w, n_phases):
    outs = tuple(jax.ShapeDtypeStruct((2, NP, w), jnp.float32)
                 for _ in range(n_phases))

    @functools.partial(
        pl.kernel,
        out_type=outs,
        mesh=mesh,
        compiler_params=_SC_PARAMS,
        scratch_types=[
            pltpu.VMEM((CH,), jnp.int32),
            pltpu.VMEM((CH,), jnp.int32),
            pltpu.VMEM((CH, w), jnp.float32),
            pltpu.VMEM((CC, w), jnp.float32),
            pltpu.VMEM((CC, w), jnp.float32),
            pltpu.VMEM_SHARED((NP, w), jnp.float32),
            pltpu.SemaphoreType.DMA,
        ],
    )
    def seg(*refs):
        tables = refs[0:n_phases]
        srcs = refs[n_phases:2 * n_phases]
        dsts = refs[2 * n_phases:3 * n_phases]
        zz = refs[3 * n_phases]
        outs_ = refs[3 * n_phases + 1:4 * n_phases + 1]
        si, di, rows, bounce, zbuf, acc, sem = refs[4 * n_phases + 1:]
        cid = lax.axis_index("c")
        sid = lax.axis_index("s")
        wid = sid * 2 + cid
        pltpu.sync_copy(zz, zbuf)
        _zero_acc(sid, acc, zbuf)
        plsc.subcore_barrier()
        for p in range(n_phases):
            _run_phase(cid, sid, wid, srcs[p], dsts[p], tables[p], acc,
                       outs_[p], si, di, rows, bounce, zbuf, sem)

    return seg


@functools.cache
def _sc_kernels():
    mesh = plsc.VectorSubcoreMesh(core_axis_name="c", subcore_axis_name="s",
                                  num_cores=2, num_subcores=16)
    seg3x4 = _make_seg(mesh, 8, 3)
    seg2x8 = _make_seg(mesh, 8, 2)
    return seg3x4, seg2x8


def _elu(h):
    return jnp.where(h > 0, h, jnp.exp(jnp.minimum(h, 0.0)) - 1.0)


def _h1_body(p_ref, x_ref, wr_ref, wo_ref, b_ref, o_ref):
    agg = p_ref[0] + p_ref[1]
    h = (jnp.dot(agg[:, :2], wr_ref[...], preferred_element_type=jnp.float32)
         + jnp.dot(x_ref[...], wo_ref[...], preferred_element_type=jnp.float32)
         + b_ref[...])
    o_ref[...] = _elu(h)


_h1_call = pl.pallas_call(
    _h1_body,
    grid=(GC,),
    in_specs=[
        pl.BlockSpec((2, RB, 8), lambda i: (0, i, 0)),
        pl.BlockSpec((RB, 2), lambda i: (i, 0)),
        pl.BlockSpec((2, 8), lambda i: (0, 0)),
        pl.BlockSpec((2, 8), lambda i: (0, 0)),
        pl.BlockSpec((1, 8), lambda i: (0, 0)),
    ],
    out_specs=pl.BlockSpec((RB, 8), lambda i: (i, 0)),
    out_shape=jax.ShapeDtypeStruct((NP, 8), jnp.float32),
)


def _h3_body(p_ref, x_ref, wr_ref, wo_ref, b_ref, oa_ref, ob_ref):
    agg = p_ref[0] + p_ref[1]
    h = (jnp.dot(agg[:, :2], wr_ref[...], preferred_element_type=jnp.float32)
         + jnp.dot(x_ref[...], wo_ref[...], preferred_element_type=jnp.float32)
         + b_ref[...])
    h = _elu(h)
    oa_ref[...] = h[:, :8]
    ob_ref[...] = h[:, 8:]


_h3_call = pl.pallas_call(
    _h3_body,
    grid=(GC,),
    in_specs=[
        pl.BlockSpec((2, RB, 8), lambda i: (0, i, 0)),
        pl.BlockSpec((RB, 2), lambda i: (i, 0)),
        pl.BlockSpec((2, 16), lambda i: (0, 0)),
        pl.BlockSpec((2, 16), lambda i: (0, 0)),
        pl.BlockSpec((1, 16), lambda i: (0, 0)),
    ],
    out_specs=[
        pl.BlockSpec((RB, 8), lambda i: (i, 0)),
        pl.BlockSpec((RB, 8), lambda i: (i, 0)),
    ],
    out_shape=[
        jax.ShapeDtypeStruct((NP, 8), jnp.float32),
        jax.ShapeDtypeStruct((NP, 8), jnp.float32),
    ],
)


def _pool_body(p_ref, h1_ref, wr_ref, wo_ref, b_ref, o_ref):
    i = pl.program_id(0)
    agg = p_ref[0] + p_ref[1]
    h2 = (jnp.dot(agg, wr_ref[...], preferred_element_type=jnp.float32)
          + jnp.dot(h1_ref[...], wo_ref[...], preferred_element_type=jnp.float32)
          + b_ref[...])
    rows = lax.broadcasted_iota(jnp.int32, (RB, 16), 0) + i * RB
    h2 = jnp.where(rows < NN, h2, -jnp.inf)
    bm = jnp.max(h2, axis=0, keepdims=True)

    @pl.when(i == 0)
    def _():
        o_ref[...] = bm

    @pl.when(i > 0)
    def _():
        o_ref[...] = jnp.maximum(o_ref[...], bm)


_pool_call = pl.pallas_call(
    _pool_body,
    grid=(GC,),
    in_specs=[
        pl.BlockSpec((2, RB, 8), lambda i: (0, i, 0)),
        pl.BlockSpec((RB, 8), lambda i: (i, 0)),
        pl.BlockSpec((8, 16), lambda i: (0, 0)),
        pl.BlockSpec((8, 16), lambda i: (0, 0)),
        pl.BlockSpec((1, 16), lambda i: (0, 0)),
    ],
    out_specs=pl.BlockSpec((1, 16), lambda i: (0, 0)),
    out_shape=jax.ShapeDtypeStruct((1, 16), jnp.float32),
)


def _final_body(p4a_ref, p4b_ref, h3a_ref, h3b_ref, pc4_ref, c_ref,
                of_ref, mf_ref, wr_ref, wo_ref, b_ref, o_ref):
    a4 = jnp.concatenate([p4a_ref[0] + p4a_ref[1],
                          p4b_ref[0] + p4b_ref[1]], axis=1)
    h3 = jnp.concatenate([h3a_ref[...], h3b_ref[...]], axis=1)
    deg = pc4_ref[0][:, 2:3] + pc4_ref[1][:, 2:3]
    wr = wr_ref[...]
    wo = wo_ref[...]
    of = of_ref[...]
    mf = mf_ref[...]
    pr_rel = (jnp.dot(of, wr[16:32], preferred_element_type=jnp.float32)
              + jnp.dot(mf, wr[32:48], preferred_element_type=jnp.float32))
    pr_root = (jnp.dot(of, wo[16:32], preferred_element_type=jnp.float32)
               + jnp.dot(mf, wo[32:48], preferred_element_type=jnp.float32))
    z = (jnp.dot(a4, wr[:16], preferred_element_type=jnp.float32)
         + jnp.dot(h3, wo[:16], preferred_element_type=jnp.float32)
         + deg * pr_rel + pr_root + b_ref[...])
    o_ref[...] = jax.nn.sigmoid(z) + c_ref[...]


_final_call = pl.pallas_call(
    _final_body,
    grid=(GC,),
    in_specs=[
        pl.BlockSpec((2, RB, 8), lambda i: (0, i, 0)),
        pl.BlockSpec((2, RB, 8), lambda i: (0, i, 0)),
        pl.BlockSpec((RB, 8), lambda i: (i, 0)),
        pl.BlockSpec((RB, 8), lambda i: (i, 0)),
        pl.BlockSpec((2, RB, 8), lambda i: (0, i, 0)),
        pl.BlockSpec((RB, 2), lambda i: (i, 0)),
        pl.BlockSpec((1, 16), lambda i: (0, 0)),
        pl.BlockSpec((1, 16), lambda i: (0, 0)),
        pl.BlockSpec((48, 2), lambda i: (0, 0)),
        pl.BlockSpec((48, 2), lambda i: (0, 0)),
        pl.BlockSpec((1, 2), lambda i: (0, 0)),
    ],
    out_specs=pl.BlockSpec((RB, 2), lambda i: (i, 0)),
    out_shape=jax.ShapeDtypeStruct((NP, 2), jnp.float32),
)


def _pad_idx(v):
    pad = EPG - EE
    fill = (jnp.arange(pad, dtype=jnp.int32) % 128) + NN
    return jnp.concatenate([v, fill])


def _pack8(x):
    one = jnp.ones((NN, 1), jnp.float32)
    return jnp.pad(jnp.concatenate([x, one], axis=1), ((0, NP - NN), (0, 5)))


def _padn(x):
    return jnp.pad(x, ((0, NP - NN), (0, 0)))


def kernel(o, m, c, edge_index_o, edge_index_m, edge_index_c,
           W1_rel, W1_root, b1, W2_rel, W2_root, b2,
           W3_rel, W3_root, b3, W4_rel, W4_root, b4):
    so, do_ = _pad_idx(edge_index_o[0]), _pad_idx(edge_index_o[1])
    sm, dm = _pad_idx(edge_index_m[0]), _pad_idx(edge_index_m[1])
    sc_, dc = _pad_idx(edge_index_c[0]), _pad_idx(edge_index_c[1])

    o_pad, m_pad, c_pad = _padn(o), _padn(m), _padn(c)
    z8 = jnp.zeros((CC, 8), jnp.float32)

    seg3x4, seg2x8 = _sc_kernels()
    p_o4, p_m4, p_c4 = seg3x4(_pack8(o), _pack8(m), _pack8(c),
                              so, sm, sc_, do_, dm, dc, z8)
    h1o = _h1_call(p_o4, o_pad, W1_rel, W1_root, b1.reshape(1, 8))
    h1m = _h1_call(p_m4, m_pad, W1_rel, W1_root, b1.reshape(1, 8))
    h3a, h3b = _h3_call(p_c4, c_pad, W3_rel, W3_root, b3.reshape(1, 16))

    p2o, p2m = seg2x8(h1o, h1m, so, sm, do_, dm, z8)
    p4a, p4b = seg2x8(h3a, h3b, sc_, sc_, dc, dc, z8)

    o_feat = _pool_call(p2o, h1o, W2_rel, W2_root, b2.reshape(1, 16))
    m_feat = _pool_call(p2m, h1m, W2_rel, W2_root, b2.reshape(1, 16))

    out = _final_call(p4a, p4b, h3a, h3b, p_c4, c_pad, o_feat, m_feat,
                      W4_rel, W4_root, b4.reshape(1, 2))
    return out[:NN]


# speedup vs baseline: 41.9273x; 1.1905x over previous
import functools

import jax
import jax.numpy as jnp
from jax import lax
from jax.experimental import pallas as pl
from jax.experimental.pallas import tpu as pltpu
from jax.experimental.pallas import tpu_sc as plsc

NN = 100000
NP = 100352
EE = 3200000
NW = 32
EW = EE // NW
CH = 2048
NF = EW // CH
TAIL = EW - NF * CH
CC = 1568
RPT = NP // 16
RB = 2048
GC = NP // RB

_SC_PARAMS = pltpu.CompilerParams(use_tc_tiling_on_sc=False)


def _zero_acc(sid, acc, zsrc):
    def body(i, carry):
        pltpu.sync_copy(zsrc, acc.at[pl.ds(sid * RPT + i * CC, CC)])
        return carry

    lax.fori_loop(0, RPT // CC, body, 0)


def _run_phase(cid, sid, wid, src_hbm, dst_hbm, x_hbm, acc, out,
               si, di, rows, isem, gsem, ssem, sit, dit, tsem, zz):
    base = wid * EW

    def idx_d(k, s):
        return (pltpu.make_async_copy(src_hbm.at[pl.ds(base + k * CH, CH)],
                                      si[s], isem[s]),
                pltpu.make_async_copy(dst_hbm.at[pl.ds(base + k * CH, CH)],
                                      di[s], isem[s]))

    def gather_d(s):
        return pltpu.make_async_copy(x_hbm.at[si[s]], rows[s], gsem[s])

    def scatter_d(s):
        return pltpu.make_async_copy(rows[s], acc.at[di[s]], ssem[s])

    def idx_start(k, s):
        a, b = idx_d(k, s)
        a.start()
        b.start()

    def idx_wait(s):
        a, b = idx_d(0, s)
        a.wait()
        b.wait()

    def step(k, wait_sc, refill):
        s = k % 3
        r = (k + 2) % 3
        if wait_sc:
            scatter_d(r).wait()
        if refill:
            idx_start(k + 2, r)
        gather_d(s).wait()
        if refill:
            idx_wait(r)
            gather_d(r).start()
        scatter_d(s).start(add=True)

    idx_start(0, 0)
    idx_wait(0)
    gather_d(0).start()
    idx_start(1, 1)
    idx_wait(1)
    gather_d(1).start()

    step(0, False, True)
    step(1, True, True)
    step(2, True, True)

    def body(t, carry):
        k0 = 3 * t

        def tstep(off):
            s = off % 3
            r = (off + 2) % 3
            scatter_d(r).wait()
            idx_start_k = base + (k0 + off + 2) * CH
            a = pltpu.make_async_copy(src_hbm.at[pl.ds(idx_start_k, CH)],
                                      si[r], isem[r])
            b = pltpu.make_async_copy(dst_hbm.at[pl.ds(idx_start_k, CH)],
                                      di[r], isem[r])
            a.start()
            b.start()
            gather_d(s).wait()
            a.wait()
            b.wait()
            gather_d(r).start()
            scatter_d(s).start(add=True)

        tstep(0)
        tstep(1)
        tstep(2)
        return carry

    lax.fori_loop(1, 15, body, 0)

    step(45, True, True)
    step(46, True, False)
    step(47, True, False)
    scatter_d(47 % 3).wait()

    r0 = base + NF * CH
    rowst = rows[2].at[pl.ds(0, TAIL)]
    pltpu.sync_copy(src_hbm.at[pl.ds(r0, TAIL)], sit)
    pltpu.sync_copy(dst_hbm.at[pl.ds(r0, TAIL)], dit)
    pltpu.async_copy(x_hbm.at[sit], rowst, tsem).wait()
    pltpu.sync_copy(rowst, acc.at[dit], add=True)
    plsc.subcore_barrier()

    bounce = rows[1].at[pl.ds(0, CC)]

    def obody(i, carry):
        off = sid * RPT + i * CC
        pltpu.sync_copy(acc.at[pl.ds(off, CC)], bounce)
        pltpu.sync_copy(bounce, out.at[cid].at[pl.ds(off, CC)])
        return carry

    lax.fori_loop(0, RPT // CC, obody, 0)
    zsrc = rows[0].at[pl.ds(0, CC)]
    pltpu.sync_copy(zz, zsrc)
    _zero_acc(sid, acc, zsrc)
    plsc.subcore_barrier()


def _make_seg(mesh, w, n_phases):
    outs = tuple(jax.ShapeDtypeStruct((2, NP, w), jnp.float32)
                 for _ in range(n_phases))

    @functools.partial(
        pl.kernel,
        out_type=outs,
        mesh=mesh,
        compiler_params=_SC_PARAMS,
        scratch_types=[
            *(pltpu.VMEM((CH,), jnp.int32) for _ in range(6)),
            *(pltpu.VMEM((CH, w), jnp.float32) for _ in range(3)),
            pltpu.VMEM((TAIL,), jnp.int32),
            pltpu.VMEM((TAIL,), jnp.int32),
            pltpu.VMEM_SHARED((NP, w), jnp.float32),
            *(pltpu.SemaphoreType.DMA for _ in range(10)),
        ],
    )
    def seg(*refs):
        tables = refs[0:n_phases]
        eis = refs[n_phases:2 * n_phases]
        zz = refs[2 * n_phases]
        outs_ = refs[2 * n_phases + 1:3 * n_phases + 1]
        sc = refs[3 * n_phases + 1:]
        si, di, rows = sc[0:3], sc[3:6], sc[6:9]
        sit, dit = sc[9:11]
        acc = sc[11]
        isem, gsem, ssem = sc[12:15], sc[15:18], sc[18:21]
        tsem = sc[21]
        cid = lax.axis_index("c")
        sid = lax.axis_index("s")
        wid = sid * 2 + cid
        zsrc = rows[0].at[pl.ds(0, CC)]
        pltpu.sync_copy(zz, zsrc)
        _zero_acc(sid, acc, zsrc)
        plsc.subcore_barrier()
        for p in range(n_phases):
            _run_phase(cid, sid, wid, eis[p].at[0], eis[p].at[1], tables[p],
                       acc, outs_[p], si, di, rows, isem, gsem, ssem,
                       sit, dit, tsem, zz)

    return seg


@functools.cache
def _sc_kernels():
    mesh = plsc.VectorSubcoreMesh(core_axis_name="c", subcore_axis_name="s",
                                  num_cores=2, num_subcores=16)
    seg3x4 = _make_seg(mesh, 8, 3)
    seg4x8 = _make_seg(mesh, 8, 4)
    return seg3x4, seg4x8


def _elu(h):
    return jnp.where(h > 0, h, jnp.exp(jnp.minimum(h, 0.0)) - 1.0)


def _h1_body(p_ref, x_ref, wr_ref, wo_ref, b_ref, o_ref):
    agg = p_ref[0] + p_ref[1]
    h = (jnp.dot(agg[:, :2], wr_ref[...], preferred_element_type=jnp.float32)
         + jnp.dot(x_ref[...], wo_ref[...], preferred_element_type=jnp.float32)
         + b_ref[...])
    o_ref[...] = _elu(h)


_h1_call = pl.pallas_call(
    _h1_body,
    grid=(GC,),
    in_specs=[
        pl.BlockSpec((2, RB, 8), lambda i: (0, i, 0)),
        pl.BlockSpec((RB, 2), lambda i: (i, 0)),
        pl.BlockSpec((2, 8), lambda i: (0, 0)),
        pl.BlockSpec((2, 8), lambda i: (0, 0)),
        pl.BlockSpec((1, 8), lambda i: (0, 0)),
    ],
    out_specs=pl.BlockSpec((RB, 8), lambda i: (i, 0)),
    out_shape=jax.ShapeDtypeStruct((NP, 8), jnp.float32),
)


def _h3_body(p_ref, x_ref, wr_ref, wo_ref, b_ref, oa_ref, ob_ref):
    agg = p_ref[0] + p_ref[1]
    h = (jnp.dot(agg[:, :2], wr_ref[...], preferred_element_type=jnp.float32)
         + jnp.dot(x_ref[...], wo_ref[...], preferred_element_type=jnp.float32)
         + b_ref[...])
    h = _elu(h)
    oa_ref[...] = h[:, :8]
    ob_ref[...] = h[:, 8:]


_h3_call = pl.pallas_call(
    _h3_body,
    grid=(GC,),
    in_specs=[
        pl.BlockSpec((2, RB, 8), lambda i: (0, i, 0)),
        pl.BlockSpec((RB, 2), lambda i: (i, 0)),
        pl.BlockSpec((2, 16), lambda i: (0, 0)),
        pl.BlockSpec((2, 16), lambda i: (0, 0)),
        pl.BlockSpec((1, 16), lambda i: (0, 0)),
    ],
    out_specs=[
        pl.BlockSpec((RB, 8), lambda i: (i, 0)),
        pl.BlockSpec((RB, 8), lambda i: (i, 0)),
    ],
    out_shape=[
        jax.ShapeDtypeStruct((NP, 8), jnp.float32),
        jax.ShapeDtypeStruct((NP, 8), jnp.float32),
    ],
)


def _pool_body(p_ref, h1_ref, wr_ref, wo_ref, b_ref, o_ref):
    i = pl.program_id(0)
    agg = p_ref[0] + p_ref[1]
    h2 = (jnp.dot(agg, wr_ref[...], preferred_element_type=jnp.float32)
          + jnp.dot(h1_ref[...], wo_ref[...], preferred_element_type=jnp.float32)
          + b_ref[...])
    rows = lax.broadcasted_iota(jnp.int32, (RB, 16), 0) + i * RB
    h2 = jnp.where(rows < NN, h2, -jnp.inf)
    bm = jnp.max(h2, axis=0, keepdims=True)

    @pl.when(i == 0)
    def _():
        o_ref[...] = bm

    @pl.when(i > 0)
    def _():
        o_ref[...] = jnp.maximum(o_ref[...], bm)


_pool_call = pl.pallas_call(
    _pool_body,
    grid=(GC,),
    in_specs=[
        pl.BlockSpec((2, RB, 8), lambda i: (0, i, 0)),
        pl.BlockSpec((RB, 8), lambda i: (i, 0)),
        pl.BlockSpec((8, 16), lambda i: (0, 0)),
        pl.BlockSpec((8, 16), lambda i: (0, 0)),
        pl.BlockSpec((1, 16), lambda i: (0, 0)),
    ],
    out_specs=pl.BlockSpec((1, 16), lambda i: (0, 0)),
    out_shape=jax.ShapeDtypeStruct((1, 16), jnp.float32),
)


def _final_body(p4a_ref, p4b_ref, h3a_ref, h3b_ref, pc4_ref, c_ref,
                of_ref, mf_ref, wr_ref, wo_ref, b_ref, o_ref):
    a4 = jnp.concatenate([p4a_ref[0] + p4a_ref[1],
                          p4b_ref[0] + p4b_ref[1]], axis=1)
    h3 = jnp.concatenate([h3a_ref[...], h3b_ref[...]], axis=1)
    deg = pc4_ref[0][:, 2:3] + pc4_ref[1][:, 2:3]
    wr = wr_ref[...]
    wo = wo_ref[...]
    of = of_ref[...]
    mf = mf_ref[...]
    pr_rel = (jnp.dot(of, wr[16:32], preferred_element_type=jnp.float32)
              + jnp.dot(mf, wr[32:48], preferred_element_type=jnp.float32))
    pr_root = (jnp.dot(of, wo[16:32], preferred_element_type=jnp.float32)
               + jnp.dot(mf, wo[32:48], preferred_element_type=jnp.float32))
    z = (jnp.dot(a4, wr[:16], preferred_element_type=jnp.float32)
         + jnp.dot(h3, wo[:16], preferred_element_type=jnp.float32)
         + deg * pr_rel + pr_root + b_ref[...])
    o_ref[...] = jax.nn.sigmoid(z) + c_ref[...]


_final_call = pl.pallas_call(
    _final_body,
    grid=(GC,),
    in_specs=[
        pl.BlockSpec((2, RB, 8), lambda i: (0, i, 0)),
        pl.BlockSpec((2, RB, 8), lambda i: (0, i, 0)),
        pl.BlockSpec((RB, 8), lambda i: (i, 0)),
        pl.BlockSpec((RB, 8), lambda i: (i, 0)),
        pl.BlockSpec((2, RB, 8), lambda i: (0, i, 0)),
        pl.BlockSpec((RB, 2), lambda i: (i, 0)),
        pl.BlockSpec((1, 16), lambda i: (0, 0)),
        pl.BlockSpec((1, 16), lambda i: (0, 0)),
        pl.BlockSpec((48, 2), lambda i: (0, 0)),
        pl.BlockSpec((48, 2), lambda i: (0, 0)),
        pl.BlockSpec((1, 2), lambda i: (0, 0)),
    ],
    out_specs=pl.BlockSpec((RB, 2), lambda i: (i, 0)),
    out_shape=jax.ShapeDtypeStruct((NP, 2), jnp.float32),
)


def _pack8(x):
    one = jnp.ones((NN, 1), jnp.float32)
    return jnp.pad(jnp.concatenate([x, one], axis=1), ((0, NP - NN), (0, 5)))


def _padn(x):
    return jnp.pad(x, ((0, NP - NN), (0, 0)))


def kernel(o, m, c, edge_index_o, edge_index_m, edge_index_c,
           W1_rel, W1_root, b1, W2_rel, W2_root, b2,
           W3_rel, W3_root, b3, W4_rel, W4_root, b4):
    o_pad, m_pad, c_pad = _padn(o), _padn(m), _padn(c)
    z8 = jnp.zeros((CC, 8), jnp.float32)

    seg3x4, seg4x8 = _sc_kernels()
    p_o4, p_m4, p_c4 = seg3x4(_pack8(o), _pack8(m), _pack8(c),
                              edge_index_o, edge_index_m, edge_index_c, z8)
    h1o = _h1_call(p_o4, o_pad, W1_rel, W1_root, b1.reshape(1, 8))
    h1m = _h1_call(p_m4, m_pad, W1_rel, W1_root, b1.reshape(1, 8))
    h3a, h3b = _h3_call(p_c4, c_pad, W3_rel, W3_root, b3.reshape(1, 16))

    p2o, p2m, p4a, p4b = seg4x8(h1o, h1m, h3a, h3b, edge_index_o,
                                edge_index_m, edge_index_c, edge_index_c, z8)

    o_feat = _pool_call(p2o, h1o, W2_rel, W2_root, b2.reshape(1, 16))
    m_feat = _pool_call(p2m, h1m, W2_rel, W2_root, b2.reshape(1, 16))

    out = _final_call(p4a, p4b, h3a, h3b, p_c4, c_pad, o_feat, m_feat,
                      W4_rel, W4_root, b4.reshape(1, 2))
    return out[:NN]
